# Initial kernel scaffold; baseline (speedup 1.0000x reference)
import jax
import jax.numpy as jnp
from jax.experimental import pallas as pl


def kernel(inputs, indexs, emb_table, W_err, b_err, W_cor, b_cor):
    raise NotImplementedError("write your pallas kernel here")


# speedup vs baseline: 1.2921x; 1.2921x over previous
import functools

import jax
import jax.numpy as jnp
from jax import lax
from jax.experimental import pallas as pl
from jax.experimental.pallas import tpu as pltpu
from jax.experimental.pallas import tpu_sc as plsc

_NC = 2
_NS = 16


def _sc_gather(emb_table, tok):
    n_tok = tok.shape[0]
    d = emb_table.shape[1]
    nw = _NC * _NS
    tpt = n_tok // nw

    mesh = plsc.VectorSubcoreMesh(core_axis_name="c", subcore_axis_name="s")

    @functools.partial(
        pl.kernel,
        out_type=jax.ShapeDtypeStruct((n_tok, d), jnp.float32),
        mesh=mesh,
        scratch_types=[
            pltpu.VMEM((tpt,), jnp.int32),
            pltpu.VMEM((tpt, d), jnp.float32),
            pltpu.SemaphoreType.DMA,
        ],
    )
    def gather(emb_hbm, tok_hbm, out_hbm, tok_v, rows_v, sem):
        wid = lax.axis_index("s") * _NC + lax.axis_index("c")
        base = wid * tpt
        pltpu.sync_copy(tok_hbm.at[pl.ds(base, tpt)], tok_v)
        pltpu.async_copy(emb_hbm.at[tok_v], rows_v, sem).wait()
        pltpu.sync_copy(rows_v, out_hbm.at[pl.ds(base, tpt)])

    return gather(emb_table, tok)


def _heads_body(idx_ref, emb_ref, w_ref, bc_ref, we_ref, be_ref,
                oc_ref, oe_ref, m_ref):
    i = pl.program_id(0)
    nb, s = idx_ref.shape

    @pl.when(i == 0)
    def _():
        for bb in range(nb):
            row = idx_ref[pl.ds(bb, 1), :]
            oh = (lax.broadcasted_iota(jnp.int32, (s, s), 0) == row
                  ).astype(jnp.float32)
            m_ref[pl.ds(bb * s, s), :] = lax.dot_general(
                oh, emb_ref[pl.ds(bb * s, s), :], (((1,), (0,)), ((), ())),
                preferred_element_type=jnp.float32)
        oe_ref[...] = lax.dot_general(
            m_ref[...], we_ref[...], (((1,), (1,)), ((), ())),
            preferred_element_type=jnp.float32) + be_ref[...]

    oc_ref[...] = lax.dot_general(
        m_ref[...], w_ref[...], (((1,), (1,)), ((), ())),
        preferred_element_type=jnp.float32) + bc_ref[...]


def _tc_heads(indexs, emb, w_cor, b_cor, w_err_pad, b_err_pad, vt=512):
    n, d = emb.shape
    nb = indexs.shape[0]
    vocab = w_cor.shape[0]
    ne = w_err_pad.shape[0]
    grid = (pl.cdiv(vocab, vt),)
    out_cor, out_err = pl.pallas_call(
        _heads_body,
        grid=grid,
        in_specs=[
            pl.BlockSpec(indexs.shape, lambda i: (0, 0)),
            pl.BlockSpec((n, d), lambda i: (0, 0)),
            pl.BlockSpec((vt, d), lambda i: (i, 0)),
            pl.BlockSpec((1, vt), lambda i: (0, i)),
            pl.BlockSpec((ne, d), lambda i: (0, 0)),
            pl.BlockSpec((1, ne), lambda i: (0, 0)),
        ],
        out_specs=[
            pl.BlockSpec((n, vt), lambda i: (0, i)),
            pl.BlockSpec((n, ne), lambda i: (0, 0)),
        ],
        out_shape=[
            jax.ShapeDtypeStruct((n, vocab), jnp.float32),
            jax.ShapeDtypeStruct((n, ne), jnp.float32),
        ],
        scratch_shapes=[pltpu.VMEM((n, d), jnp.float32)],
        compiler_params=pltpu.CompilerParams(
            dimension_semantics=("arbitrary",)),
    )(indexs, emb, w_cor, b_cor.reshape(1, vocab), w_err_pad,
      b_err_pad.reshape(1, ne))
    return out_cor, out_err


def kernel(inputs, indexs, emb_table, W_err, b_err, W_cor, b_cor):
    b, s = inputs.shape
    vocab, d = emb_table.shape
    n_err = W_err.shape[0]

    tok = inputs.reshape(-1).astype(jnp.int32)
    emb = _sc_gather(emb_table, tok)

    ne_pad = 128
    w_err_pad = jnp.zeros((ne_pad, d), jnp.float32).at[:n_err].set(W_err)
    b_err_pad = jnp.zeros((ne_pad,), jnp.float32).at[:n_err].set(b_err)

    out_cor, out_err = _tc_heads(indexs.astype(jnp.int32), emb, W_cor, b_cor,
                                 w_err_pad, b_err_pad)
    return (out_err[:, :n_err].reshape(b, s, n_err),
            out_cor.reshape(b, s, vocab))
